# baseline (speedup 1.0000x reference)
import functools

import jax
import jax.numpy as jnp
from jax import lax
from jax.experimental import pallas as pl
from jax.experimental.pallas import tpu as pltpu
from jax.experimental.pallas import tpu_sc as plsc

_NUM_CORES = 2
_NUM_SUBCORES = 16
_NUM_WORKERS = _NUM_CORES * _NUM_SUBCORES
_CHUNK = 16
_NGBUF = 4
_NPBUF = 2


def _sc_gather(table, idx_flat, alpha, beta):
    n_idx = idx_flat.shape[0]
    hidden = table.shape[1]
    per_worker = n_idx // _NUM_WORKERS
    mesh = plsc.VectorSubcoreMesh(core_axis_name="c", subcore_axis_name="s")

    @functools.partial(
        pl.kernel,
        out_type=jax.ShapeDtypeStruct((n_idx, hidden), table.dtype),
        mesh=mesh,
        scratch_types=[
            pltpu.VMEM((per_worker,), jnp.int32),
            pltpu.VMEM((hidden,), table.dtype),
            pltpu.VMEM((hidden,), table.dtype),
            [pltpu.VMEM((_CHUNK, hidden), table.dtype)] * _NGBUF,
            [pltpu.VMEM((_CHUNK, hidden), table.dtype)] * _NPBUF,
            [pltpu.SemaphoreType.DMA] * _NGBUF,
            [pltpu.SemaphoreType.DMA] * _NPBUF,
        ],
    )
    def kern(table_hbm, idx_hbm, alpha_hbm, beta_hbm, out_hbm,
             idx_v, alpha_v, beta_v, gbufs, pbufs, sem_g, sem_o):
        wid = lax.axis_index("s") * _NUM_CORES + lax.axis_index("c")
        base = wid * per_worker
        pltpu.sync_copy(idx_hbm.at[pl.ds(base, per_worker)], idx_v)
        pltpu.sync_copy(alpha_hbm, alpha_v)
        pltpu.sync_copy(beta_hbm, beta_v)

        def gather(c, b):
            return pltpu.async_copy(
                table_hbm.at[idx_v.at[pl.ds(c, _CHUNK)]], gbufs[b], sem_g[b]
            )

        def gather_wait(c, b):
            pltpu.make_async_copy(
                table_hbm.at[idx_v.at[pl.ds(c, _CHUNK)]], gbufs[b], sem_g[b]
            ).wait()

        def put(c, p):
            return pltpu.async_copy(
                pbufs[p], out_hbm.at[pl.ds(base + c, _CHUNK)], sem_o[p]
            )

        def put_wait(c, p):
            pltpu.make_async_copy(
                pbufs[p], out_hbm.at[pl.ds(base + c, _CHUNK)], sem_o[p]
            ).wait()

        def affine(b, p):
            src = gbufs[b]
            dst = pbufs[p]

            @plsc.parallel_loop(0, hidden, step=16, unroll=16)
            def _(h):
                a = alpha_v[pl.ds(h, 16)]
                bb = beta_v[pl.ds(h, 16)]
                for r in range(_CHUNK):
                    dst[r, pl.ds(h, 16)] = src[r, pl.ds(h, 16)] * a + bb

        gather(0, 0)
        gather(_CHUNK, 1)
        gather(2 * _CHUNK, 2)
        gather(3 * _CHUNK, 3)
        gather_wait(0, 0)
        affine(0, 0)
        put(0, 0)
        gather_wait(_CHUNK, 1)
        affine(1, 1)
        put(_CHUNK, 1)

        @pl.loop(2 * _CHUNK, per_worker - 2 * _CHUNK, step=_NGBUF * _CHUNK)
        def _(c):
            for b in range(_NGBUF):
                bb = (b + 2) % _NGBUF
                pp = b % _NPBUF
                cur = c + b * _CHUNK
                gather(cur + 2 * _CHUNK, b)
                put_wait(cur - 2 * _CHUNK, pp)
                gather_wait(cur, bb)
                affine(bb, pp)
                put(cur, pp)

        tail = per_worker - 2 * _CHUNK
        put_wait(tail - 2 * _CHUNK, 0)
        gather_wait(tail, 2)
        affine(2, 0)
        put(tail, 0)
        put_wait(tail - _CHUNK, 1)
        gather_wait(tail + _CHUNK, 3)
        affine(3, 1)
        put(tail + _CHUNK, 1)
        put_wait(tail, 0)
        put_wait(tail + _CHUNK, 1)

    return kern(table, idx_flat, alpha, beta)


def kernel(position_ids, pe, alpha, beta):
    batch, seq = position_ids.shape
    hidden = pe.shape[1]
    out = _sc_gather(pe, position_ids.reshape(batch * seq), alpha, beta)
    return out.reshape(batch, seq, hidden)


# speedup vs baseline: 1.0229x; 1.0229x over previous
import functools

import jax
import jax.numpy as jnp
from jax import lax
from jax.experimental import pallas as pl
from jax.experimental.pallas import tpu as pltpu
from jax.experimental.pallas import tpu_sc as plsc

_NUM_CORES = 2
_NUM_SUBCORES = 16
_NUM_WORKERS = _NUM_CORES * _NUM_SUBCORES
_CHUNK = 16
_NGBUF = 4
_NPBUF = 2


def _sc_gather(table, idx_flat, alpha, beta):
    n_idx = idx_flat.shape[0]
    hidden = table.shape[1]
    per_worker = n_idx // _NUM_WORKERS
    mesh = plsc.VectorSubcoreMesh(core_axis_name="c", subcore_axis_name="s")

    @functools.partial(
        pl.kernel,
        out_type=jax.ShapeDtypeStruct((n_idx, hidden), table.dtype),
        mesh=mesh,
        scratch_types=[
            pltpu.VMEM((per_worker,), jnp.int32),
            pltpu.VMEM((hidden,), table.dtype),
            pltpu.VMEM((hidden,), table.dtype),
            [pltpu.VMEM((_CHUNK, hidden), table.dtype)] * _NGBUF,
            [pltpu.VMEM((_CHUNK, hidden), table.dtype)] * _NPBUF,
            [pltpu.SemaphoreType.DMA] * _NGBUF,
            [pltpu.SemaphoreType.DMA] * _NPBUF,
        ],
    )
    def kern(table_hbm, idx_hbm, alpha_hbm, beta_hbm, out_hbm,
             idx_v, alpha_v, beta_v, gbufs, pbufs, sem_g, sem_o):
        wid = lax.axis_index("s") * _NUM_CORES + lax.axis_index("c")
        base = wid * per_worker
        pltpu.sync_copy(idx_hbm.at[pl.ds(base, per_worker)], idx_v)
        pltpu.sync_copy(alpha_hbm, alpha_v)
        pltpu.sync_copy(beta_hbm, beta_v)

        def gather(c, b):
            return pltpu.async_copy(
                table_hbm.at[idx_v.at[pl.ds(c, _CHUNK)]], gbufs[b], sem_g[b]
            )

        def gather_wait(c, b):
            pltpu.make_async_copy(
                table_hbm.at[idx_v.at[pl.ds(c, _CHUNK)]], gbufs[b], sem_g[b]
            ).wait()

        def put(c, p):
            return pltpu.async_copy(
                pbufs[p], out_hbm.at[pl.ds(base + c, _CHUNK)], sem_o[p]
            )

        def put_wait(c, p):
            pltpu.make_async_copy(
                pbufs[p], out_hbm.at[pl.ds(base + c, _CHUNK)], sem_o[p]
            ).wait()

        def affine(b, p):
            src = gbufs[b]
            dst = pbufs[p]

            @plsc.parallel_loop(0, hidden, step=16, unroll=8)
            def _(h):
                a = alpha_v[pl.ds(h, 16)]
                bb = beta_v[pl.ds(h, 16)]
                for r in range(_CHUNK):
                    dst[r, pl.ds(h, 16)] = src[r, pl.ds(h, 16)] * a + bb

        gather(0, 0)
        gather(_CHUNK, 1)
        gather(2 * _CHUNK, 2)
        gather(3 * _CHUNK, 3)
        gather_wait(0, 0)
        affine(0, 0)
        put(0, 0)
        gather_wait(_CHUNK, 1)
        affine(1, 1)
        put(_CHUNK, 1)

        @pl.loop(2 * _CHUNK, per_worker - 2 * _CHUNK, step=_NGBUF * _CHUNK)
        def _(c):
            for b in range(_NGBUF):
                bb = (b + 2) % _NGBUF
                pp = b % _NPBUF
                cur = c + b * _CHUNK
                gather(cur + 2 * _CHUNK, b)
                put_wait(cur - 2 * _CHUNK, pp)
                gather_wait(cur, bb)
                affine(bb, pp)
                put(cur, pp)

        tail = per_worker - 2 * _CHUNK
        put_wait(tail - 2 * _CHUNK, 0)
        gather_wait(tail, 2)
        affine(2, 0)
        put(tail, 0)
        put_wait(tail - _CHUNK, 1)
        gather_wait(tail + _CHUNK, 3)
        affine(3, 1)
        put(tail + _CHUNK, 1)
        put_wait(tail, 0)
        put_wait(tail + _CHUNK, 1)

    return kern(table, idx_flat, alpha, beta)


def kernel(position_ids, pe, alpha, beta):
    batch, seq = position_ids.shape
    hidden = pe.shape[1]
    out = _sc_gather(pe, position_ids.reshape(batch * seq), alpha, beta)
    return out.reshape(batch, seq, hidden)
